# Initial kernel scaffold; baseline (speedup 1.0000x reference)
import jax
import jax.numpy as jnp
from jax.experimental import pallas as pl


def kernel(fine, gt_points, x2_d, W1, b1, W2, b2, W3, b3, epoch, is_training):
    raise NotImplementedError("write your pallas kernel here")


# speedup vs baseline: 5.3202x; 5.3202x over previous
import functools

import jax
import jax.numpy as jnp
from jax import lax
from jax.experimental import pallas as pl
from jax.experimental.pallas import tpu as pltpu
from jax.experimental.pallas import tpu_sc as plsc

B, N_GT, N_FINE = 16, 8192, 2048


def _make_fps(nb, ngt, nfine, interpret=False):
    chunks = ngt // 16

    def _fps_body(gt_hbm, samp_hbm, xv, yv, zv, dv, sx, sy, sz):
        cid = lax.axis_index("c")
        sid = lax.axis_index("s")
        wid = cid * 16 + sid

        @pl.when(wid < nb)
        def _():
            b = wid
            pltpu.sync_copy(gt_hbm.at[pl.ds((b * 3 + 0) * ngt, ngt)], xv)
            pltpu.sync_copy(gt_hbm.at[pl.ds((b * 3 + 1) * ngt, ngt)], yv)
            pltpu.sync_copy(gt_hbm.at[pl.ds((b * 3 + 2) * ngt, ngt)], zv)

            lanes = lax.iota(jnp.int32, 16)
            zeros_i = jnp.zeros((16,), jnp.int32)
            lane0 = lanes == 0
            big = jnp.full((16,), jnp.float32(jnp.inf))
            fzero = jnp.float32(0.0)

            def init_chunk(r, _):
                dv[pl.ds(r * 16, 16)] = big
                return 0

            lax.fori_loop(0, chunks, init_chunk, 0)

            def splat_at(ref, idx):
                base = (idx // 16) * 16
                v = ref[pl.ds(base, 16)]
                sel = lanes == (idx - base)
                return jnp.full((16,), jnp.sum(jnp.where(sel, v, fzero)))

            px = splat_at(xv, jnp.int32(0))
            py = splat_at(yv, jnp.int32(0))
            pz = splat_at(zv, jnp.int32(0))
            pendx = jnp.where(lane0, px, fzero)
            pendy = jnp.where(lane0, py, fzero)
            pendz = jnp.where(lane0, pz, fzero)

            neg = jnp.full((16,), jnp.float32(-jnp.inf))

            def step(j, carry):
                px, py, pz, pendx, pendy, pendz = carry

                def chunk(r, c2):
                    bv, bi = c2
                    sl = pl.ds(r * 16, 16)
                    dx = xv[sl] - px
                    dy = yv[sl] - py
                    dz = zv[sl] - pz
                    d = (dx * dx + dy * dy) + dz * dz
                    dn = jnp.minimum(dv[sl], d)
                    dv[sl] = dn
                    upd = dn > bv
                    bv = jnp.where(upd, dn, bv)
                    bi = jnp.where(upd, lanes + r * 16, bi)
                    return bv, bi

                bv, bi = lax.fori_loop(0, chunks, chunk, (neg, zeros_i))
                m = jnp.max(bv)
                cand = jnp.where(bv == m, bi, jnp.int32(2**30))
                nxt = jnp.min(cand)
                npx = splat_at(xv, nxt)
                npy = splat_at(yv, nxt)
                npz = splat_at(zv, nxt)
                t = j % 16
                put = lanes == t
                pendx = jnp.where(put, npx, pendx)
                pendy = jnp.where(put, npy, pendy)
                pendz = jnp.where(put, npz, pendz)

                @pl.when(t == 15)
                def _flush():
                    blk = pl.ds((j // 16) * 16, 16)
                    sx[blk] = pendx
                    sy[blk] = pendy
                    sz[blk] = pendz

                return npx, npy, npz, pendx, pendy, pendz

            lax.fori_loop(1, nfine, step, (px, py, pz, pendx, pendy, pendz))

            pltpu.sync_copy(sx, samp_hbm.at[pl.ds((b * 3 + 0) * nfine, nfine)])
            pltpu.sync_copy(sy, samp_hbm.at[pl.ds((b * 3 + 1) * nfine, nfine)])
            pltpu.sync_copy(sz, samp_hbm.at[pl.ds((b * 3 + 2) * nfine, nfine)])

    return functools.partial(
        pl.kernel,
        mesh=plsc.VectorSubcoreMesh(core_axis_name="c", subcore_axis_name="s"),
        out_type=jax.ShapeDtypeStruct((nb * 3 * nfine,), jnp.float32),
        scratch_types=[
            pltpu.VMEM((ngt,), jnp.float32),
            pltpu.VMEM((ngt,), jnp.float32),
            pltpu.VMEM((ngt,), jnp.float32),
            pltpu.VMEM((ngt,), jnp.float32),
            pltpu.VMEM((nfine,), jnp.float32),
            pltpu.VMEM((nfine,), jnp.float32),
            pltpu.VMEM((nfine,), jnp.float32),
        ],
        compiler_params=pltpu.CompilerParams(needs_layout_passes=False),
        interpret=interpret,
    )(_fps_body)


_fps_kernel = _make_fps(B, N_GT, N_FINE)


def kernel(fine, gt_points, x2_d, W1, b1, W2, b2, W3, b3, epoch, is_training):
    gt_T = jnp.transpose(gt_points, (0, 2, 1)).reshape(-1)
    sampled_T = _fps_kernel(gt_T).reshape(B, 3, N_FINE)
    gt_sampled = jnp.transpose(sampled_T, (0, 2, 1))

    fine_t = jnp.transpose(fine, (0, 2, 1))

    a2 = jnp.sum(fine_t * fine_t, axis=-1)
    b2_ = jnp.sum(gt_sampled * gt_sampled, axis=-1)
    d = a2[:, :, None] + b2_[:, None, :] - 2.0 * jnp.einsum(
        'bnd,bmd->bnm', fine_t, gt_sampled)
    d = jnp.maximum(d, 0.0)
    dist1 = jnp.min(d, axis=2)
    confidence_score = jnp.exp(-dist1)

    h = jax.nn.gelu(jnp.einsum('oc,bcn->bon', W1, x2_d) + b1[None, :, None],
                    approximate=False)
    h = jax.nn.gelu(jnp.einsum('oc,bcn->bon', W2, h) + b2[None, :, None],
                    approximate=False)
    pred = jnp.einsum('oc,bcn->bon', W3, h) + b3[None, :, None]
    pred_sq = jnp.squeeze(pred, axis=1)
    confidence_score_loss = jnp.mean((pred_sq - confidence_score) ** 2)

    cs = jnp.where(is_training != 0, confidence_score, pred_sq)
    order = jnp.argsort(-cs, axis=1)
    fine_sorted = jnp.take_along_axis(fine_t, order[:, :, None], axis=1)
    filtered_fine = jnp.where(epoch < 50, fine_t, fine_sorted)
    return (filtered_fine, confidence_score_loss)


# speedup vs baseline: 5.3218x; 1.0003x over previous
import functools

import jax
import jax.numpy as jnp
from jax import lax
from jax.experimental import pallas as pl
from jax.experimental.pallas import tpu as pltpu
from jax.experimental.pallas import tpu_sc as plsc

B, N_GT, N_FINE = 16, 8192, 2048


def _make_fps(nb, ngt, nfine, interpret=False):
    chunks = ngt // 16

    def _fps_body(gt_hbm, samp_hbm, xv, yv, zv, dv, sx, sy, sz):
        cid = lax.axis_index("c")
        sid = lax.axis_index("s")
        wid = cid * 16 + sid

        @pl.when(wid < nb)
        def _():
            b = wid
            pltpu.sync_copy(gt_hbm.at[pl.ds((b * 3 + 0) * ngt, ngt)], xv)
            pltpu.sync_copy(gt_hbm.at[pl.ds((b * 3 + 1) * ngt, ngt)], yv)
            pltpu.sync_copy(gt_hbm.at[pl.ds((b * 3 + 2) * ngt, ngt)], zv)

            lanes = lax.iota(jnp.int32, 16)
            zeros_i = jnp.zeros((16,), jnp.int32)
            lane0 = lanes == 0
            big = jnp.full((16,), jnp.float32(jnp.inf))
            fzero = jnp.float32(0.0)

            def init_chunk(r, _):
                dv[pl.ds(r * 16, 16)] = big
                return 0

            lax.fori_loop(0, chunks, init_chunk, 0)

            def splat_at(ref, idx):
                base = (idx // 16) * 16
                v = ref[pl.ds(base, 16)]
                sel = lanes == (idx - base)
                return jnp.full((16,), jnp.sum(jnp.where(sel, v, fzero)))

            px = splat_at(xv, jnp.int32(0))
            py = splat_at(yv, jnp.int32(0))
            pz = splat_at(zv, jnp.int32(0))
            pendx = jnp.where(lane0, px, fzero)
            pendy = jnp.where(lane0, py, fzero)
            pendz = jnp.where(lane0, pz, fzero)

            neg = jnp.full((16,), jnp.float32(-jnp.inf))

            def step(j, carry):
                px, py, pz, pendx, pendy, pendz = carry

                def chunk(rb, c2):
                    bv, bi = c2
                    for u in range(8):
                        sl = pl.ds(rb * 128 + u * 16, 16)
                        dx = xv[sl] - px
                        dy = yv[sl] - py
                        dz = zv[sl] - pz
                        d = (dx * dx + dy * dy) + dz * dz
                        dn = jnp.minimum(dv[sl], d)
                        dv[sl] = dn
                        upd = dn > bv
                        bv = jnp.where(upd, dn, bv)
                        bi = jnp.where(upd, lanes + (rb * 128 + u * 16), bi)
                    return bv, bi

                bv, bi = lax.fori_loop(0, chunks // 8, chunk, (neg, zeros_i))
                m = jnp.max(bv)
                cand = jnp.where(bv == m, bi, jnp.int32(2**30))
                nxt = jnp.min(cand)
                npx = splat_at(xv, nxt)
                npy = splat_at(yv, nxt)
                npz = splat_at(zv, nxt)
                t = j % 16
                put = lanes == t
                pendx = jnp.where(put, npx, pendx)
                pendy = jnp.where(put, npy, pendy)
                pendz = jnp.where(put, npz, pendz)

                @pl.when(t == 15)
                def _flush():
                    blk = pl.ds((j // 16) * 16, 16)
                    sx[blk] = pendx
                    sy[blk] = pendy
                    sz[blk] = pendz

                return npx, npy, npz, pendx, pendy, pendz

            lax.fori_loop(1, nfine, step, (px, py, pz, pendx, pendy, pendz))

            pltpu.sync_copy(sx, samp_hbm.at[pl.ds((b * 3 + 0) * nfine, nfine)])
            pltpu.sync_copy(sy, samp_hbm.at[pl.ds((b * 3 + 1) * nfine, nfine)])
            pltpu.sync_copy(sz, samp_hbm.at[pl.ds((b * 3 + 2) * nfine, nfine)])

    return functools.partial(
        pl.kernel,
        mesh=plsc.VectorSubcoreMesh(core_axis_name="c", subcore_axis_name="s"),
        out_type=jax.ShapeDtypeStruct((nb * 3 * nfine,), jnp.float32),
        scratch_types=[
            pltpu.VMEM((ngt,), jnp.float32),
            pltpu.VMEM((ngt,), jnp.float32),
            pltpu.VMEM((ngt,), jnp.float32),
            pltpu.VMEM((ngt,), jnp.float32),
            pltpu.VMEM((nfine,), jnp.float32),
            pltpu.VMEM((nfine,), jnp.float32),
            pltpu.VMEM((nfine,), jnp.float32),
        ],
        compiler_params=pltpu.CompilerParams(needs_layout_passes=False),
        interpret=interpret,
    )(_fps_body)


_fps_kernel = _make_fps(B, N_GT, N_FINE)


def kernel(fine, gt_points, x2_d, W1, b1, W2, b2, W3, b3, epoch, is_training):
    gt_T = jnp.transpose(gt_points, (0, 2, 1)).reshape(-1)
    sampled_T = _fps_kernel(gt_T).reshape(B, 3, N_FINE)
    gt_sampled = jnp.transpose(sampled_T, (0, 2, 1))

    fine_t = jnp.transpose(fine, (0, 2, 1))

    a2 = jnp.sum(fine_t * fine_t, axis=-1)
    b2_ = jnp.sum(gt_sampled * gt_sampled, axis=-1)
    d = a2[:, :, None] + b2_[:, None, :] - 2.0 * jnp.einsum(
        'bnd,bmd->bnm', fine_t, gt_sampled)
    d = jnp.maximum(d, 0.0)
    dist1 = jnp.min(d, axis=2)
    confidence_score = jnp.exp(-dist1)

    h = jax.nn.gelu(jnp.einsum('oc,bcn->bon', W1, x2_d) + b1[None, :, None],
                    approximate=False)
    h = jax.nn.gelu(jnp.einsum('oc,bcn->bon', W2, h) + b2[None, :, None],
                    approximate=False)
    pred = jnp.einsum('oc,bcn->bon', W3, h) + b3[None, :, None]
    pred_sq = jnp.squeeze(pred, axis=1)
    confidence_score_loss = jnp.mean((pred_sq - confidence_score) ** 2)

    cs = jnp.where(is_training != 0, confidence_score, pred_sq)
    order = jnp.argsort(-cs, axis=1)
    fine_sorted = jnp.take_along_axis(fine_t, order[:, :, None], axis=1)
    filtered_fine = jnp.where(epoch < 50, fine_t, fine_sorted)
    return (filtered_fine, confidence_score_loss)


# speedup vs baseline: 18.2530x; 3.4299x over previous
import functools

import jax
import jax.numpy as jnp
from jax import lax
from jax.experimental import pallas as pl
from jax.experimental.pallas import tpu as pltpu
from jax.experimental.pallas import tpu_sc as plsc

B, N_GT, N_FINE = 16, 8192, 2048


def _make_fps(nb, ngt, nfine, interpret=False):
    chunks = ngt // 16

    def _fps_body(gt_hbm, samp_hbm, xv, yv, zv, dv, sx, sy, sz):
        cid = lax.axis_index("c")
        sid = lax.axis_index("s")
        wid = cid * 16 + sid

        @pl.when(wid < nb)
        def _():
            b = wid
            pltpu.sync_copy(gt_hbm.at[pl.ds((b * 3 + 0) * ngt, ngt)], xv)
            pltpu.sync_copy(gt_hbm.at[pl.ds((b * 3 + 1) * ngt, ngt)], yv)
            pltpu.sync_copy(gt_hbm.at[pl.ds((b * 3 + 2) * ngt, ngt)], zv)

            lanes = lax.iota(jnp.int32, 16)
            zeros_i = jnp.zeros((16,), jnp.int32)
            lane0 = lanes == 0
            big = jnp.full((16,), jnp.float32(jnp.inf))
            fzero = jnp.float32(0.0)

            def init_chunk(r, _):
                dv[pl.ds(r * 16, 16)] = big
                return 0

            lax.fori_loop(0, chunks, init_chunk, 0)

            def splat_at(ref, idx):
                base = (idx // 16) * 16
                v = ref[pl.ds(base, 16)]
                sel = lanes == (idx - base)
                return jnp.full((16,), jnp.sum(jnp.where(sel, v, fzero)))

            px = splat_at(xv, jnp.int32(0))
            py = splat_at(yv, jnp.int32(0))
            pz = splat_at(zv, jnp.int32(0))
            pendx = jnp.where(lane0, px, fzero)
            pendy = jnp.where(lane0, py, fzero)
            pendz = jnp.where(lane0, pz, fzero)

            neg = jnp.full((16,), jnp.float32(-jnp.inf))

            def step(j, carry):
                px, py, pz, pendx, pendy, pendz = carry

                @plsc.parallel_loop(0, chunks, step=1, unroll=8,
                                    carry=(neg, zeros_i))
                def chunk(r, c2):
                    bv, bi = c2
                    sl = pl.ds(r * 16, 16)
                    dx = xv[sl] - px
                    dy = yv[sl] - py
                    dz = zv[sl] - pz
                    d = (dx * dx + dy * dy) + dz * dz
                    dn = jnp.minimum(dv[sl], d)
                    dv[sl] = dn
                    upd = dn > bv
                    bv = jnp.where(upd, dn, bv)
                    bi = jnp.where(upd, lanes + r * 16, bi)
                    return bv, bi

                bv, bi = chunk
                m = jnp.max(bv)
                cand = jnp.where(bv == m, bi, jnp.int32(2**30))
                nxt = jnp.min(cand)
                npx = splat_at(xv, nxt)
                npy = splat_at(yv, nxt)
                npz = splat_at(zv, nxt)
                t = j % 16
                put = lanes == t
                pendx = jnp.where(put, npx, pendx)
                pendy = jnp.where(put, npy, pendy)
                pendz = jnp.where(put, npz, pendz)

                @pl.when(t == 15)
                def _flush():
                    blk = pl.ds((j // 16) * 16, 16)
                    sx[blk] = pendx
                    sy[blk] = pendy
                    sz[blk] = pendz

                return npx, npy, npz, pendx, pendy, pendz

            lax.fori_loop(1, nfine, step, (px, py, pz, pendx, pendy, pendz))

            pltpu.sync_copy(sx, samp_hbm.at[pl.ds((b * 3 + 0) * nfine, nfine)])
            pltpu.sync_copy(sy, samp_hbm.at[pl.ds((b * 3 + 1) * nfine, nfine)])
            pltpu.sync_copy(sz, samp_hbm.at[pl.ds((b * 3 + 2) * nfine, nfine)])

    return functools.partial(
        pl.kernel,
        mesh=plsc.VectorSubcoreMesh(core_axis_name="c", subcore_axis_name="s"),
        out_type=jax.ShapeDtypeStruct((nb * 3 * nfine,), jnp.float32),
        scratch_types=[
            pltpu.VMEM((ngt,), jnp.float32),
            pltpu.VMEM((ngt,), jnp.float32),
            pltpu.VMEM((ngt,), jnp.float32),
            pltpu.VMEM((ngt,), jnp.float32),
            pltpu.VMEM((nfine,), jnp.float32),
            pltpu.VMEM((nfine,), jnp.float32),
            pltpu.VMEM((nfine,), jnp.float32),
        ],
        compiler_params=pltpu.CompilerParams(needs_layout_passes=False),
        interpret=interpret,
    )(_fps_body)


_fps_kernel = _make_fps(B, N_GT, N_FINE)


def kernel(fine, gt_points, x2_d, W1, b1, W2, b2, W3, b3, epoch, is_training):
    gt_T = jnp.transpose(gt_points, (0, 2, 1)).reshape(-1)
    sampled_T = _fps_kernel(gt_T).reshape(B, 3, N_FINE)
    gt_sampled = jnp.transpose(sampled_T, (0, 2, 1))

    fine_t = jnp.transpose(fine, (0, 2, 1))

    a2 = jnp.sum(fine_t * fine_t, axis=-1)
    b2_ = jnp.sum(gt_sampled * gt_sampled, axis=-1)
    d = a2[:, :, None] + b2_[:, None, :] - 2.0 * jnp.einsum(
        'bnd,bmd->bnm', fine_t, gt_sampled)
    d = jnp.maximum(d, 0.0)
    dist1 = jnp.min(d, axis=2)
    confidence_score = jnp.exp(-dist1)

    h = jax.nn.gelu(jnp.einsum('oc,bcn->bon', W1, x2_d) + b1[None, :, None],
                    approximate=False)
    h = jax.nn.gelu(jnp.einsum('oc,bcn->bon', W2, h) + b2[None, :, None],
                    approximate=False)
    pred = jnp.einsum('oc,bcn->bon', W3, h) + b3[None, :, None]
    pred_sq = jnp.squeeze(pred, axis=1)
    confidence_score_loss = jnp.mean((pred_sq - confidence_score) ** 2)

    cs = jnp.where(is_training != 0, confidence_score, pred_sq)
    order = jnp.argsort(-cs, axis=1)
    fine_sorted = jnp.take_along_axis(fine_t, order[:, :, None], axis=1)
    filtered_fine = jnp.where(epoch < 50, fine_t, fine_sorted)
    return (filtered_fine, confidence_score_loss)
